# baseline (speedup 1.0000x reference)
import functools

import jax
import jax.numpy as jnp
from jax import lax
from jax.experimental import pallas as pl
from jax.experimental.pallas import tpu as pltpu
from jax.experimental.pallas import tpu_sc as plsc

_ALPHA = 0.1

_NC = 2
_NS = 16
_NW = _NC * _NS
_K = 128
_G = 4
_ROW = 128


def _leaky(x):
    return jnp.where(x > 0, x, _ALPHA * x)


def _tc_table(atom_features, W_cat, n_rows):
    n_atoms, d_atom = atom_features.shape
    d_cat = W_cat.shape[1]

    def body(af_ref, wcat_ref, t_ref):
        t_ref[0:n_atoms, :] = jnp.dot(af_ref[...], wcat_ref[...],
                                      preferred_element_type=jnp.float32)
        t_ref[n_atoms:n_rows, :] = jnp.zeros(
            (n_rows - n_atoms, d_cat), jnp.float32)

    return pl.pallas_call(
        body,
        out_shape=jax.ShapeDtypeStruct((n_rows, d_cat), jnp.float32),
    )(atom_features, W_cat)


def _tc_tp(pair_features, W_pap_p, b_pap, pair_block):
    n_pairs, d_pair = pair_features.shape
    d_agg = W_pap_p.shape[1]
    grid = n_pairs // pair_block

    def body(pf_ref, wpp_ref, bpap_ref, tp_ref):
        tp_ref[...] = (
            jnp.dot(pf_ref[...], wpp_ref[...],
                    preferred_element_type=jnp.float32) + bpap_ref[...])

    return pl.pallas_call(
        body,
        grid=(grid,),
        in_specs=[
            pl.BlockSpec((pair_block, d_pair), lambda i: (i, 0)),
            pl.BlockSpec((d_pair, d_agg), lambda i: (0, 0)),
            pl.BlockSpec((1, d_agg), lambda i: (0, 0)),
        ],
        out_specs=pl.BlockSpec((pair_block, d_agg), lambda i: (i, 0)),
        out_shape=jax.ShapeDtypeStruct((n_pairs, d_agg), jnp.float32),
    )(pair_features, W_pap_p, b_pap)


def _sc_gather_scatter(pair_i, pair_j, T, Tp_flat, d_agg, d_out, n_real):
    n_pairs = pair_i.shape[0]
    n_atoms, d_cat = T.shape
    n_chunks = n_pairs // _K
    n_real_chunks = n_real // _K
    per_sub = n_chunks // _NS
    n_groups = per_sub // _G
    assert n_pairs % _K == 0 and n_chunks % _NS == 0 and per_sub % _G == 0
    assert n_real % _K == 0 and _G % 2 == 0 and n_chunks % 2 == 0
    H = -(-n_atoms // (_NC * _NS * 8)) * (_NS * 8)
    rows_per_sub = H // _NS
    acc_rows = H + 8

    mesh = plsc.VectorSubcoreMesh(core_axis_name="c", subcore_axis_name="s")

    @functools.partial(
        pl.kernel,
        out_type=[
            jax.ShapeDtypeStruct((_NC * H, _ROW), jnp.float32),
            jax.ShapeDtypeStruct((n_real * d_out,), jnp.float32),
        ],
        mesh=mesh,
        scratch_types=[
            pltpu.VMEM((_G * _K,), jnp.int32),
            pltpu.VMEM((_G * _K,), jnp.int32),
            pltpu.VMEM((_K,), jnp.int32),
            pltpu.VMEM((_K,), jnp.int32),
            pltpu.VMEM((_K, _ROW), jnp.float32),
            pltpu.VMEM((_K, _ROW), jnp.float32),
            pltpu.VMEM((_K, _ROW), jnp.float32),
            pltpu.VMEM((_K, _ROW), jnp.float32),
            pltpu.VMEM((2 * _K * d_agg,), jnp.float32),
            pltpu.VMEM((_K * d_out,), jnp.float32),
            pltpu.VMEM_SHARED((acc_rows, _ROW), jnp.float32),
            pltpu.SemaphoreType.DMA,
            pltpu.SemaphoreType.DMA,
            pltpu.SemaphoreType.DMA,
            pltpu.SemaphoreType.DMA,
            pltpu.SemaphoreType.DMA,
            pltpu.SemaphoreType.DMA,
            pltpu.SemaphoreType.DMA,
            pltpu.SemaphoreType.DMA,
        ],
    )
    def sc_kernel(pi_hbm, pj_hbm, t_hbm, tp_hbm,
                  s_out, vv_out,
                  idx_ig, idx_jg, idx_s0, idx_s1,
                  ti0, ti1, tj0, tj1, buf_tp, vvb, s_sh,
                  sem_tp, sem_ti0, sem_ti1, sem_tj0, sem_tj1,
                  sem_sc0, sem_sc1, sem_vv):
        cid = lax.axis_index("c")
        sid = lax.axis_index("s")
        ti = (ti0, ti1)
        tj = (tj0, tj1)
        idx_s = (idx_s0, idx_s1)
        sem_ti = (sem_ti0, sem_ti1)
        sem_tj = (sem_tj0, sem_tj1)
        sem_sc = (sem_sc0, sem_sc1)

        zero16 = jnp.zeros((16,), jnp.float32)
        row_lo = cid * H

        def zrow(r, carry):
            for cc in range(_ROW // 16):
                ti0[r, pl.ds(cc * 16, 16)] = zero16
            return carry

        lax.fori_loop(0, _K, zrow, 0)
        z0 = sid * rows_per_sub
        off = 0
        while off < rows_per_sub + 8:
            sz = min(_K, rows_per_sub + 8 - off)
            pltpu.sync_copy(ti0.at[pl.ds(0, sz)],
                            s_sh.at[pl.ds(z0 + off, sz)])
            off += sz
        plsc.subcore_barrier()

        c_first = sid * per_sub

        def group(u, carry):
            g0 = c_first + u * _G
            pbase = g0 * _K
            pltpu.sync_copy(pi_hbm.at[pl.ds(pbase, _G * _K)], idx_ig)
            pltpu.sync_copy(pj_hbm.at[pl.ds(pbase, _G * _K)], idx_jg)
            h_tp = pltpu.async_copy(
                tp_hbm.at[pl.ds(pbase * d_agg, 2 * _K * d_agg)],
                buf_tp, sem_tp)

            def issue_gathers(g):
                p = g % 2
                hti = pltpu.async_copy(
                    t_hbm.at[idx_ig.at[pl.ds(g * _K, _K)]], ti[p], sem_ti[p])
                htj = pltpu.async_copy(
                    t_hbm.at[idx_jg.at[pl.ds(g * _K, _K)]], tj[p], sem_tj[p])
                return hti, htj

            h_g = [None] * _G
            h_sc = [None] * _G
            h_vv = [None] * _G
            h_tp2 = [None]
            h_g[0] = issue_gathers(0)

            for g in range(_G):
                p = g % 2
                tib, tjb, isp = ti[p], tj[p], idx_s[p]
                own_vv = (((g0 + g) % 2) == cid) & (g0 + g < n_real_chunks)

                if g + 1 < _G:
                    if h_sc[g - 1] is not None:
                        h_sc[g - 1].wait()
                        h_sc[g - 1] = None
                    h_g[g + 1] = issue_gathers(g + 1)

                if g == 0:
                    h_tp.wait()
                if g == 2:
                    h_tp2[0].wait()

                h_g[g][0].wait()
                h_g[g][1].wait()

                for k in range(_K // 16):
                    sl = pl.ds(g * _K + k * 16, 16)
                    v = idx_ig[sl] - row_lo
                    ok = (v >= 0) & (v < H)
                    isp[pl.ds(k * 16, 16)] = jnp.where(ok, v, H)

                tp_half = (g % 2) * _K * d_agg

                def arow(r4, c2, tib=tib, tjb=tjb, tp_half=tp_half):
                    for rr in range(4):
                        r = r4 * 4 + rr
                        for cc in range(d_agg // 16):
                            a = (tib[r, pl.ds(cc * 16, 16)]
                                 + tjb[r, pl.ds(d_agg + cc * 16, 16)]
                                 + buf_tp[pl.ds(tp_half + r * d_agg
                                                + cc * 16, 16)])
                            tib[r, pl.ds(cc * 16, 16)] = jnp.where(
                                a > 0, a, _ALPHA * a)
                    return c2

                lax.fori_loop(0, _K // 4, arow, 0)
                h_sc[g] = pltpu.async_copy(tib, s_sh.at[isp], sem_sc[p],
                                           add=True)

                if g == 1:
                    h_tp2[0] = pltpu.async_copy(
                        tp_hbm.at[pl.ds((pbase + 2 * _K) * d_agg,
                                        2 * _K * d_agg)],
                        buf_tp, sem_tp)

                vv_dst = vv_out.at[pl.ds((g0 + g) * _K * d_out, _K * d_out)]
                h_vv[g] = pltpu.make_async_copy(vvb, vv_dst, sem_vv)

                @pl.when(own_vv)
                def _(g=g, tib=tib, tjb=tjb):
                    if g - 2 >= 0:
                        h_vv[g - 2].wait()

                    def vrow(r4, c2):
                        for rr in range(4):
                            r = r4 * 4 + rr
                            for cc in range(d_out // 16):
                                sl = pl.ds(2 * d_agg + cc * 16, 16)
                                vvb[pl.ds(r * d_out + cc * 16, 16)] = (
                                    tib[r, sl] + tjb[r, sl])
                        return c2

                    lax.fori_loop(0, _K // 4, vrow, 0)
                    h_vv[g].start()

            for g in range(_G):
                if h_sc[g] is not None:
                    h_sc[g].wait()

            for g in (_G - 2, _G - 1):
                @pl.when((((g0 + g) % 2) == cid)
                         & (g0 + g < n_real_chunks))
                def _(g=g):
                    h_vv[g].wait()

            return carry

        lax.fori_loop(0, n_groups, group, 0)

        plsc.subcore_barrier()
        r0 = sid * rows_per_sub
        off = 0
        while off < rows_per_sub:
            sz = min(_K, rows_per_sub - off)
            pltpu.sync_copy(s_sh.at[pl.ds(r0 + off, sz)],
                            ti0.at[pl.ds(0, sz)])
            pltpu.sync_copy(ti0.at[pl.ds(0, sz)],
                            s_out.at[pl.ds(cid * H + r0 + off, sz)])
            off += sz

    return sc_kernel(pair_i, pair_j, T, Tp_flat)


def _tc_atom(atom_features, s_part, W_pa_a, W_pa_s, b_pa, W_aa, b_aa,
             W_ao1, W_ao2, b_ao):
    n_atoms, d_atom = atom_features.shape
    d_agg = W_pa_s.shape[0]
    d_out = W_aa.shape[1]

    def body(af_ref, sp_ref, wpaa_ref, wpas_ref, bpa_ref, waa_ref, baa_ref,
             wao1_ref, wao2_ref, bao_ref, out_ref):
        af = af_ref[...]
        s = sp_ref[0:n_atoms, 0:d_agg]
        a_pa = _leaky(
            jnp.dot(af, wpaa_ref[...], preferred_element_type=jnp.float32)
            + jnp.dot(s, wpas_ref[...], preferred_element_type=jnp.float32)
            + bpa_ref[...])
        a_aa = _leaky(
            jnp.dot(af, waa_ref[...], preferred_element_type=jnp.float32)
            + baa_ref[...])
        out_ref[...] = _leaky(
            jnp.dot(a_pa, wao1_ref[...], preferred_element_type=jnp.float32)
            + jnp.dot(a_aa, wao2_ref[...], preferred_element_type=jnp.float32)
            + bao_ref[...])

    return pl.pallas_call(
        body,
        out_shape=jax.ShapeDtypeStruct((n_atoms, d_out), jnp.float32),
    )(atom_features, s_part, W_pa_a, W_pa_s, b_pa, W_aa, b_aa,
      W_ao1, W_ao2, b_ao)


def _tc_pair(pair_features, vv, W_in, b_in, W_po, b_po, pair_block):
    n_pairs, d_pair = pair_features.shape
    d_out2 = W_in.shape[1]
    d_out = d_out2 // 2
    grid = n_pairs // pair_block

    def body(pf_ref, vv_ref, win_ref, bin_ref, wpo_ref, bpo_ref, out_ref):
        p = jnp.dot(pf_ref[...], win_ref[...],
                    preferred_element_type=jnp.float32) + bin_ref[...]
        p_apa = _leaky(p[:, :d_out] + vv_ref[...])
        p_pp = _leaky(p[:, d_out:])
        cat = jnp.concatenate([p_apa, p_pp], axis=1)
        out_ref[...] = _leaky(
            jnp.dot(cat, wpo_ref[...], preferred_element_type=jnp.float32)
            + bpo_ref[...])

    return pl.pallas_call(
        body,
        grid=(grid,),
        in_specs=[
            pl.BlockSpec((pair_block, d_pair), lambda i: (i, 0)),
            pl.BlockSpec((pair_block, d_out), lambda i: (i, 0)),
            pl.BlockSpec(W_in.shape, lambda i: (0, 0)),
            pl.BlockSpec((1, d_out2), lambda i: (0, 0)),
            pl.BlockSpec(W_po.shape, lambda i: (0, 0)),
            pl.BlockSpec((1, d_out), lambda i: (0, 0)),
        ],
        out_specs=pl.BlockSpec((pair_block, d_out), lambda i: (i, 0)),
        out_shape=jax.ShapeDtypeStruct((n_pairs, d_out), jnp.float32),
    )(pair_features, vv, W_in, b_in, W_po, b_po)


def kernel(atom_features, pair_features, pair_split, atom_to_pair, num_atoms,
           W_pap, b_pap, W_pa, b_pa, W_aa, b_aa, W_ao, b_ao,
           W_ap, b_ap, W_pp, b_pp, W_po, b_po):
    del pair_split, num_atoms
    n_atoms, d_atom = atom_features.shape
    n_pairs, d_pair = pair_features.shape
    d_agg = W_pap.shape[1]
    d_out_a = W_pa.shape[1]
    d_out_p = W_pp.shape[1]

    pair_i = atom_to_pair[:, 0]
    pair_j = atom_to_pair[:, 1]

    W_pap_i = W_pap[:d_atom]
    W_pap_p = W_pap[d_atom:d_atom + d_pair]
    W_pap_j = W_pap[d_atom + d_pair:]
    W_pa_a = W_pa[:d_atom]
    W_pa_s = W_pa[d_atom:]
    W_ao1 = W_ao[:d_out_a]
    W_ao2 = W_ao[d_out_a:]
    W_ap_p = W_ap[:d_pair]
    W_ap_a = W_ap[d_pair:]
    W_po1 = W_po[:d_out_p]
    W_po2 = W_po[d_out_p:]

    W_cat = jnp.concatenate([W_pap_i, W_pap_j, W_ap_a], axis=1)

    pair_block = 16000

    n_chunks_pad = -(-n_pairs // (_K * _NS * _G)) * (_NS * _G)
    n_pairs_pad = n_chunks_pad * _K
    t_rows = -(-(n_atoms + 16) // 16) * 16
    pad_idx = jnp.full((n_pairs_pad - n_pairs,), n_atoms + 8, jnp.int32)
    pair_i_pad = jnp.concatenate([pair_i, pad_idx])
    pair_j_pad = jnp.concatenate([pair_j, pad_idx])
    pf_pad = jnp.concatenate(
        [pair_features,
         jnp.zeros((n_pairs_pad - n_pairs, d_pair), jnp.float32)])

    t = _tc_table(atom_features, W_cat, t_rows)
    tp = _tc_tp(pf_pad, W_pap_p, b_pap.reshape(1, -1), n_pairs_pad // 20)

    s_part, vv_flat = _sc_gather_scatter(pair_i_pad, pair_j_pad, t,
                                         tp.reshape(-1), d_agg, d_out_p,
                                         n_pairs)
    vv = vv_flat.reshape(n_pairs, d_out_p)

    atom_hidden = _tc_atom(atom_features, s_part, W_pa_a, W_pa_s,
                           b_pa.reshape(1, -1), W_aa, b_aa.reshape(1, -1),
                           W_ao1, W_ao2, b_ao.reshape(1, -1))

    W_in = jnp.concatenate([W_ap_p, W_pp], axis=1)
    b_in = jnp.concatenate([b_ap, b_pp]).reshape(1, -1)
    pair_hidden = _tc_pair(pair_features, vv, W_in, b_in, W_po,
                           b_po.reshape(1, -1), pair_block)

    return (atom_hidden, pair_hidden)


# speedup vs baseline: 1.7956x; 1.7956x over previous
import functools

import jax
import jax.numpy as jnp
from jax import lax
from jax.experimental import pallas as pl
from jax.experimental.pallas import tpu as pltpu
from jax.experimental.pallas import tpu_sc as plsc

_ALPHA = 0.1

_NC = 2
_NS = 16
_NW = _NC * _NS
_K = 80
_G = 5
_ROW = 128


def _leaky(x):
    return jnp.where(x > 0, x, _ALPHA * x)


def _tc_table(atom_features, W_cat):
    n_atoms, d_atom = atom_features.shape
    d_cat = W_cat.shape[1]

    def body(af_ref, wcat_ref, t_ref):
        t_ref[...] = jnp.dot(af_ref[...], wcat_ref[...],
                             preferred_element_type=jnp.float32)

    return pl.pallas_call(
        body,
        out_shape=jax.ShapeDtypeStruct((n_atoms, d_cat), jnp.float32),
    )(atom_features, W_cat)


def _tc_tp(pair_features, W_pap_p, b_pap, pair_block):
    n_pairs, d_pair = pair_features.shape
    d_agg = W_pap_p.shape[1]
    grid = n_pairs // pair_block

    def body(pf_ref, wpp_ref, bpap_ref, tp_ref):
        tp_ref[...] = (
            jnp.dot(pf_ref[...], wpp_ref[...],
                    preferred_element_type=jnp.float32) + bpap_ref[...])

    return pl.pallas_call(
        body,
        grid=(grid,),
        in_specs=[
            pl.BlockSpec((pair_block, d_pair), lambda i: (i, 0)),
            pl.BlockSpec((d_pair, d_agg), lambda i: (0, 0)),
            pl.BlockSpec((1, d_agg), lambda i: (0, 0)),
        ],
        out_specs=pl.BlockSpec((pair_block, d_agg), lambda i: (i, 0)),
        out_shape=jax.ShapeDtypeStruct((n_pairs, d_agg), jnp.float32),
    )(pair_features, W_pap_p, b_pap)


def _sc_gather_scatter(pair_i, pair_j, T, Tp_flat, d_agg, d_out):
    n_pairs = pair_i.shape[0]
    n_atoms, d_cat = T.shape
    n_chunks = n_pairs // _K
    per_sub = n_chunks // _NS
    n_groups = per_sub // _G
    assert n_pairs % _K == 0 and n_chunks % _NS == 0 and per_sub % _G == 0
    assert n_chunks % (2 * _NS) == 0
    H = -(-n_atoms // (_NC * _NS * 8)) * (_NS * 8)
    rows_per_sub = H // _NS
    acc_rows = H + 8

    mesh = plsc.VectorSubcoreMesh(core_axis_name="c", subcore_axis_name="s")

    @functools.partial(
        pl.kernel,
        out_type=[
            jax.ShapeDtypeStruct((_NC * H, _ROW), jnp.float32),
            jax.ShapeDtypeStruct((n_pairs * d_out,), jnp.float32),
        ],
        mesh=mesh,
        scratch_types=[
            pltpu.VMEM((_G * _K,), jnp.int32),
            pltpu.VMEM((_G * _K,), jnp.int32),
            pltpu.VMEM((_K,), jnp.int32),
            pltpu.VMEM((_K,), jnp.int32),
            pltpu.VMEM((_K,), jnp.int32),
            pltpu.VMEM((_K, _ROW), jnp.float32),
            pltpu.VMEM((_K, _ROW), jnp.float32),
            pltpu.VMEM((_K, _ROW), jnp.float32),
            pltpu.VMEM((_K, _ROW), jnp.float32),
            pltpu.VMEM((_K, _ROW), jnp.float32),
            pltpu.VMEM((_K, _ROW), jnp.float32),
            pltpu.VMEM((_G * _K * d_agg,), jnp.float32),
            pltpu.VMEM((_K * d_out,), jnp.float32),
            pltpu.VMEM_SHARED((acc_rows, _ROW), jnp.float32),
            pltpu.SemaphoreType.DMA,
            pltpu.SemaphoreType.DMA,
            pltpu.SemaphoreType.DMA,
            pltpu.SemaphoreType.DMA,
            pltpu.SemaphoreType.DMA,
            pltpu.SemaphoreType.DMA,
            pltpu.SemaphoreType.DMA,
            pltpu.SemaphoreType.DMA,
            pltpu.SemaphoreType.DMA,
            pltpu.SemaphoreType.DMA,
            pltpu.SemaphoreType.DMA,
        ],
    )
    def sc_kernel(pi_hbm, pj_hbm, t_hbm, tp_hbm,
                  s_out, vv_out,
                  idx_ig, idx_jg, idx_s0, idx_s1, idx_s2,
                  ti0, ti1, ti2, tj0, tj1, tj2, buf_tp, vvb, s_sh,
                  sem_tp, sem_ti0, sem_ti1, sem_ti2,
                  sem_tj0, sem_tj1, sem_tj2,
                  sem_sc0, sem_sc1, sem_sc2, sem_vv):
        cid = lax.axis_index("c")
        sid = lax.axis_index("s")
        ti = (ti0, ti1, ti2)
        tj = (tj0, tj1, tj2)
        idx_s = (idx_s0, idx_s1, idx_s2)
        sem_ti = (sem_ti0, sem_ti1, sem_ti2)
        sem_tj = (sem_tj0, sem_tj1, sem_tj2)
        sem_sc = (sem_sc0, sem_sc1, sem_sc2)

        zero16 = jnp.zeros((16,), jnp.float32)
        row_lo = cid * H

        def zrow(r, carry):
            for cc in range(_ROW // 16):
                ti0[r, pl.ds(cc * 16, 16)] = zero16
            return carry

        lax.fori_loop(0, _K, zrow, 0)
        z0 = sid * rows_per_sub
        off = 0
        while off < rows_per_sub + 8:
            sz = min(_K, rows_per_sub + 8 - off)
            pltpu.sync_copy(ti0.at[pl.ds(0, sz)],
                            s_sh.at[pl.ds(z0 + off, sz)])
            off += sz
        plsc.subcore_barrier()

        c_first = sid * per_sub

        def group(u, carry):
            g0 = c_first + u * _G
            pbase = g0 * _K
            pltpu.sync_copy(pi_hbm.at[pl.ds(pbase, _G * _K)], idx_ig)
            pltpu.sync_copy(pj_hbm.at[pl.ds(pbase, _G * _K)], idx_jg)
            h_tp = pltpu.async_copy(
                tp_hbm.at[pl.ds(pbase * d_agg, _G * _K * d_agg)],
                buf_tp, sem_tp)

            def issue_gathers(g):
                p = g % 3
                hti = pltpu.async_copy(
                    t_hbm.at[idx_ig.at[pl.ds(g * _K, _K)]], ti[p], sem_ti[p])
                htj = pltpu.async_copy(
                    t_hbm.at[idx_jg.at[pl.ds(g * _K, _K)]], tj[p], sem_tj[p])
                return hti, htj

            h_g = [None] * _G
            h_sc = [None] * _G
            h_vv = [None] * _G
            h_g[0] = issue_gathers(0)
            if _G > 1:
                h_g[1] = issue_gathers(1)
            h_tp.wait()

            for g in range(_G):
                p = g % 3
                tib, tjb, isp = ti[p], tj[p], idx_s[p]
                own_vv = ((g0 + g) % 2) == cid

                if g + 2 < _G:
                    if h_sc[g - 1] is not None:
                        h_sc[g - 1].wait()
                        h_sc[g - 1] = None
                    h_g[g + 2] = issue_gathers(g + 2)

                h_g[g][0].wait()
                h_g[g][1].wait()

                for k in range(_K // 16):
                    sl = pl.ds(g * _K + k * 16, 16)
                    v = idx_ig[sl] - row_lo
                    ok = (v >= 0) & (v < H)
                    isp[pl.ds(k * 16, 16)] = jnp.where(ok, v, H)

                def arow(r4, c2):
                    for rr in range(4):
                        r = r4 * 4 + rr
                        for cc in range(d_agg // 16):
                            a = (tib[r, pl.ds(cc * 16, 16)]
                                 + tjb[r, pl.ds(d_agg + cc * 16, 16)]
                                 + buf_tp[pl.ds(g * _K * d_agg + r * d_agg
                                                + cc * 16, 16)])
                            tib[r, pl.ds(cc * 16, 16)] = jnp.where(
                                a > 0, a, _ALPHA * a)
                    return c2

                lax.fori_loop(0, _K // 4, arow, 0)
                h_sc[g] = pltpu.async_copy(tib, s_sh.at[isp], sem_sc[p],
                                           add=True)

                vv_dst = vv_out.at[pl.ds((g0 + g) * _K * d_out, _K * d_out)]
                h_vv[g] = pltpu.make_async_copy(vvb, vv_dst, sem_vv)

                @pl.when(own_vv)
                def _(g=g, tib=tib, tjb=tjb):
                    if g - 2 >= 0:
                        h_vv[g - 2].wait()

                    def vrow(r4, c2):
                        for rr in range(4):
                            r = r4 * 4 + rr
                            for cc in range(d_out // 16):
                                sl = pl.ds(2 * d_agg + cc * 16, 16)
                                vvb[pl.ds(r * d_out + cc * 16, 16)] = (
                                    tib[r, sl] + tjb[r, sl])
                        return c2

                    lax.fori_loop(0, _K // 4, vrow, 0)
                    h_vv[g].start()

            for g in range(_G):
                if h_sc[g] is not None:
                    h_sc[g].wait()

            for g in (_G - 2, _G - 1):
                @pl.when(((g0 + g) % 2) == cid)
                def _(g=g):
                    h_vv[g].wait()

            return carry

        lax.fori_loop(0, n_groups, group, 0)

        plsc.subcore_barrier()
        r0 = sid * rows_per_sub
        off = 0
        while off < rows_per_sub:
            sz = min(_K, rows_per_sub - off)
            pltpu.sync_copy(s_sh.at[pl.ds(r0 + off, sz)],
                            ti0.at[pl.ds(0, sz)])
            pltpu.sync_copy(ti0.at[pl.ds(0, sz)],
                            s_out.at[pl.ds(cid * H + r0 + off, sz)])
            off += sz

    return sc_kernel(pair_i, pair_j, T, Tp_flat)


def _tc_atom(atom_features, s_part, W_pa_a, W_pa_s, b_pa, W_aa, b_aa,
             W_ao1, W_ao2, b_ao):
    n_atoms, d_atom = atom_features.shape
    d_agg = W_pa_s.shape[0]
    d_out = W_aa.shape[1]

    def body(af_ref, sp_ref, wpaa_ref, wpas_ref, bpa_ref, waa_ref, baa_ref,
             wao1_ref, wao2_ref, bao_ref, out_ref):
        af = af_ref[...]
        s = sp_ref[0:n_atoms, 0:d_agg]
        a_pa = _leaky(
            jnp.dot(af, wpaa_ref[...], preferred_element_type=jnp.float32)
            + jnp.dot(s, wpas_ref[...], preferred_element_type=jnp.float32)
            + bpa_ref[...])
        a_aa = _leaky(
            jnp.dot(af, waa_ref[...], preferred_element_type=jnp.float32)
            + baa_ref[...])
        out_ref[...] = _leaky(
            jnp.dot(a_pa, wao1_ref[...], preferred_element_type=jnp.float32)
            + jnp.dot(a_aa, wao2_ref[...], preferred_element_type=jnp.float32)
            + bao_ref[...])

    return pl.pallas_call(
        body,
        out_shape=jax.ShapeDtypeStruct((n_atoms, d_out), jnp.float32),
    )(atom_features, s_part, W_pa_a, W_pa_s, b_pa, W_aa, b_aa,
      W_ao1, W_ao2, b_ao)


def _tc_pair(pair_features, vv, W_in, b_in, W_po, b_po, pair_block):
    n_pairs, d_pair = pair_features.shape
    d_out2 = W_in.shape[1]
    d_out = d_out2 // 2
    grid = n_pairs // pair_block

    def body(pf_ref, vv_ref, win_ref, bin_ref, wpo_ref, bpo_ref, out_ref):
        p = jnp.dot(pf_ref[...], win_ref[...],
                    preferred_element_type=jnp.float32) + bin_ref[...]
        p_apa = _leaky(p[:, :d_out] + vv_ref[...])
        p_pp = _leaky(p[:, d_out:])
        cat = jnp.concatenate([p_apa, p_pp], axis=1)
        out_ref[...] = _leaky(
            jnp.dot(cat, wpo_ref[...], preferred_element_type=jnp.float32)
            + bpo_ref[...])

    return pl.pallas_call(
        body,
        grid=(grid,),
        in_specs=[
            pl.BlockSpec((pair_block, d_pair), lambda i: (i, 0)),
            pl.BlockSpec((pair_block, d_out), lambda i: (i, 0)),
            pl.BlockSpec(W_in.shape, lambda i: (0, 0)),
            pl.BlockSpec((1, d_out2), lambda i: (0, 0)),
            pl.BlockSpec(W_po.shape, lambda i: (0, 0)),
            pl.BlockSpec((1, d_out), lambda i: (0, 0)),
        ],
        out_specs=pl.BlockSpec((pair_block, d_out), lambda i: (i, 0)),
        out_shape=jax.ShapeDtypeStruct((n_pairs, d_out), jnp.float32),
    )(pair_features, vv, W_in, b_in, W_po, b_po)


def kernel(atom_features, pair_features, pair_split, atom_to_pair, num_atoms,
           W_pap, b_pap, W_pa, b_pa, W_aa, b_aa, W_ao, b_ao,
           W_ap, b_ap, W_pp, b_pp, W_po, b_po):
    del pair_split, num_atoms
    n_atoms, d_atom = atom_features.shape
    n_pairs, d_pair = pair_features.shape
    d_agg = W_pap.shape[1]
    d_out_a = W_pa.shape[1]
    d_out_p = W_pp.shape[1]

    pair_i = atom_to_pair[:, 0]
    pair_j = atom_to_pair[:, 1]

    W_pap_i = W_pap[:d_atom]
    W_pap_p = W_pap[d_atom:d_atom + d_pair]
    W_pap_j = W_pap[d_atom + d_pair:]
    W_pa_a = W_pa[:d_atom]
    W_pa_s = W_pa[d_atom:]
    W_ao1 = W_ao[:d_out_a]
    W_ao2 = W_ao[d_out_a:]
    W_ap_p = W_ap[:d_pair]
    W_ap_a = W_ap[d_pair:]
    W_po1 = W_po[:d_out_p]
    W_po2 = W_po[d_out_p:]

    W_cat = jnp.concatenate([W_pap_i, W_pap_j, W_ap_a], axis=1)

    pair_block = 16000

    t = _tc_table(atom_features, W_cat)
    tp = _tc_tp(pair_features, W_pap_p, b_pap.reshape(1, -1), pair_block)

    s_part, vv_flat = _sc_gather_scatter(pair_i, pair_j, t,
                                         tp.reshape(-1), d_agg, d_out_p)
    vv = vv_flat.reshape(n_pairs, d_out_p)

    atom_hidden = _tc_atom(atom_features, s_part, W_pa_a, W_pa_s,
                           b_pa.reshape(1, -1), W_aa, b_aa.reshape(1, -1),
                           W_ao1, W_ao2, b_ao.reshape(1, -1))

    W_in = jnp.concatenate([W_ap_p, W_pp], axis=1)
    b_in = jnp.concatenate([b_ap, b_pp]).reshape(1, -1)
    pair_hidden = _tc_pair(pair_features, vv, W_in, b_in, W_po,
                           b_po.reshape(1, -1), pair_block)

    return (atom_hidden, pair_hidden)
